# baseline (speedup 1.0000x reference)
import numpy as np
import jax
import jax.numpy as jnp
from jax import lax
from jax.experimental import pallas as pl
from jax.experimental.pallas import tpu as pltpu
from jax.experimental.pallas import tpu_sc as plsc

_EPS = 0.1
_ROWS = 64
_COLS = 1_000_000
_NW = 32
_HALF = 500_000
_TCHUNK = 31_360
_LANES = 16
_UNROLL = 8
_TINY = np.float32(np.finfo(np.float32).tiny)
_LN2 = np.float32(0.6931471805599453)
_SQRT2 = np.float32(1.4142135623730951)
_INT_MAX = np.int32(2**31 - 1)

_TC_LANES = 125
_TC_SUB = 64
_TC_STEPS = 8000 // _TC_SUB


def _np_threefry2x32(k0, k1, x0, x1):
    u32 = np.uint32
    x0 = np.asarray(x0, dtype=u32).copy()
    x1 = np.asarray(x1, dtype=u32).copy()
    ks = [u32(k0), u32(k1), u32(u32(k0) ^ u32(k1) ^ u32(0x1BD11BDA))]
    rotations = [[13, 15, 26, 6], [17, 29, 16, 24]]
    x0 = (x0 + ks[0]).astype(u32)
    x1 = (x1 + ks[1]).astype(u32)
    for i in range(5):
        for r in rotations[i % 2]:
            x0 = (x0 + x1).astype(u32)
            x1 = ((x1 << u32(r)) | (x1 >> u32(32 - r))).astype(u32)
            x1 = (x1 ^ x0).astype(u32)
        x0 = (x0 + ks[(i + 1) % 3]).astype(u32)
        x1 = (x1 + ks[(i + 2) % 3] + u32(i + 1)).astype(u32)
    return x0, x1


def _derive_constants():
    kd1 = _np_threefry2x32(0, 42, [0], [0])
    kd2 = _np_threefry2x32(0, 42, [0], [1])
    kd1 = (int(kd1[0][0]), int(kd1[1][0]))
    kd2 = (int(kd2[0][0]), int(kd2[1][0]))
    o0, o1 = _np_threefry2x32(kd2[0], kd2[1],
                              np.zeros(_ROWS, np.uint32),
                              np.arange(_ROWS, dtype=np.uint32))
    bits = (o0 ^ o1).astype(np.uint32)
    u = (((bits >> np.uint32(9)) | np.uint32(0x3F800000))
         .view(np.float32) - np.float32(1.0))
    sampled = np.where(u < np.float32(_EPS))[0]
    return kd1, tuple(int(r) for r in sampled)


_KD1, _SAMPLED_ROWS = _derive_constants()
_NS = len(_SAMPLED_ROWS)
assert _NS >= 1


def _threefry_bits(n_u32):
    k0, k1 = _KD1
    ks0 = np.uint32(k0)
    ks1 = np.uint32(k1)
    ks2 = np.uint32(ks0 ^ ks1 ^ np.uint32(0x1BD11BDA))
    ks = [ks0, ks1, ks2]
    rotations = [[13, 15, 26, 6], [17, 29, 16, 24]]
    x0 = jnp.full(n_u32.shape, ks0, jnp.uint32)
    x1 = n_u32 + ks1
    for i in range(5):
        for r in rotations[i % 2]:
            x0 = x0 + x1
            x1 = (x1 << np.uint32(r)) | (x1 >> np.uint32(32 - r))
            x1 = x1 ^ x0
        x0 = x0 + ks[(i + 1) % 3]
        x1 = x1 + np.uint32((int(ks[(i + 2) % 3]) + i + 1) & 0xFFFFFFFF)
    return x0 ^ x1


def _log_f32(u):
    bits = lax.bitcast_convert_type(u, jnp.uint32)
    e = (bits >> np.uint32(23)).astype(jnp.int32) - 127
    m = lax.bitcast_convert_type(
        (bits & np.uint32(0x007FFFFF)) | np.uint32(0x3F800000), jnp.float32)
    big = m >= _SQRT2
    m = jnp.where(big, m * np.float32(0.5), m)
    e = e + jnp.where(big, 1, 0)
    s = (m - np.float32(1.0)) / (m + np.float32(1.0))
    z = s * s
    p = np.float32(2.0 / 7.0) + z * np.float32(2.0 / 9.0)
    p = np.float32(2.0 / 5.0) + z * p
    p = np.float32(2.0 / 3.0) + z * p
    p = np.float32(2.0) + z * p
    return e.astype(jnp.float32) * _LN2 + s * p


def _sc_body(input_hbm, pv_hbm, pi_hbm,
             smA, smB, tbufA, tbufB, stage_v, stage_i, semA, semB):
    cid = lax.axis_index("c")
    sid = lax.axis_index("s")
    wrow = cid * 16 + sid
    off_t = pl.multiple_of(jnp.minimum(sid * _TCHUNK, _HALF - _TCHUNK), 8)
    iota = lax.broadcasted_iota(jnp.int32, (_LANES,), 0)

    def hbm_copy(step, buf, sem):
        r = 32 * cid + (step // 2)
        hoff = pl.multiple_of((step % 2) * _HALF, 16)
        return pltpu.make_async_copy(
            input_hbm.at[r, pl.ds(hoff, _HALF)], buf, sem)

    def pull(sbuf, tbuf):
        pltpu.sync_copy(sbuf.at[pl.ds(off_t, _TCHUNK)], tbuf)

    def row_scan(tbuf, colbase):
        def body(i, carry):
            vbs, ibs, gs = carry
            vbs, ibs, gs = list(vbs), list(ibs), list(gs)
            base = i * (_UNROLL * _LANES)
            for j in range(_UNROLL):
                v = tbuf[pl.ds(base + j * _LANES, _LANES)]
                m = v > vbs[j]
                vbs[j] = jnp.maximum(vbs[j], v)
                ibs[j] = jnp.where(m, gs[j], ibs[j])
                gs[j] = gs[j] + (_UNROLL * _LANES)
            return tuple(vbs), tuple(ibs), tuple(gs)

        vbs = tuple(jnp.full((_LANES,), -1.0, jnp.float32)
                    for _ in range(_UNROLL))
        ibs = tuple(jnp.zeros((_LANES,), jnp.int32) for _ in range(_UNROLL))
        gs = tuple(colbase + j * _LANES + iota for j in range(_UNROLL))
        vbs, ibs, _ = lax.fori_loop(0, _TCHUNK // (_UNROLL * _LANES), body,
                                    (vbs, ibs, gs))
        vm = vbs[0]
        for j in range(1, _UNROLL):
            vm = jnp.maximum(vm, vbs[j])
        mx = jnp.max(vm)
        best = _INT_MAX * jnp.ones((_LANES,), jnp.int32)
        for j in range(_UNROLL):
            best = jnp.minimum(
                best, jnp.where(vbs[j] == mx, ibs[j], _INT_MAX))
        return mx, jnp.min(best)

    is0 = sid == 0

    @pl.when(is0)
    def _():
        hbm_copy(0, smA, semA).start()
        hbm_copy(1, smB, semB).start()

    def half(step, sbuf, tbuf, sem, colbase):
        @pl.when(is0)
        def _():
            hbm_copy(step, sbuf, sem).wait()
        plsc.subcore_barrier()
        pull(sbuf, tbuf)
        plsc.subcore_barrier()

        @pl.when(is0)
        def _():
            hbm_copy((step + 2) & 63, sbuf, sem).start()
        return row_scan(tbuf, colbase)

    for grp in range(2):
        def body(j16, carry, grp=grp):
            av, ai = carry
            k0 = 2 * (grp * 16 + j16)
            v0, i0 = half(k0, smA, tbufA, semA, off_t)
            v1, i1 = half(k0 + 1, smB, tbufB, semB, _HALF + off_t)
            upd = v1 > v0
            val = jnp.where(upd, v1, v0)
            idx = jnp.where(upd, i1, i0)
            m = iota == j16
            av = jnp.where(m, jnp.full((_LANES,), val, jnp.float32), av)
            ai = jnp.where(m, jnp.full((_LANES,), idx, jnp.int32), ai)
            return av, ai
        av = jnp.zeros((_LANES,), jnp.float32)
        ai = jnp.zeros((_LANES,), jnp.int32)
        av, ai = lax.fori_loop(0, 16, body, (av, ai))
        stage_v[pl.ds(grp * 16, _LANES)] = av
        stage_i[pl.ds(grp * 16, _LANES)] = ai

    @pl.when(is0)
    def _():
        hbm_copy(0, smA, semA).wait()
        hbm_copy(1, smB, semB).wait()
    pltpu.sync_copy(stage_v, pv_hbm.at[wrow])
    pltpu.sync_copy(stage_i, pi_hbm.at[wrow])


_sc_call = pl.kernel(
    _sc_body,
    out_type=(jax.ShapeDtypeStruct((_NW, 32), jnp.float32),
              jax.ShapeDtypeStruct((_NW, 32), jnp.int32)),
    mesh=plsc.VectorSubcoreMesh(core_axis_name="c", subcore_axis_name="s",
                                num_cores=2, num_subcores=16),
    scratch_types=[
        pltpu.VMEM_SHARED((_HALF,), jnp.float32),
        pltpu.VMEM_SHARED((_HALF,), jnp.float32),
        pltpu.VMEM((_TCHUNK,), jnp.float32),
        pltpu.VMEM((_TCHUNK,), jnp.float32),
        pltpu.VMEM((32,), jnp.float32),
        pltpu.VMEM((32,), jnp.int32),
        pltpu.SemaphoreType.DMA,
        pltpu.SemaphoreType.DMA,
    ],
    compiler_params=pltpu.CompilerParams(use_tc_tiling_on_sc=False,
                                         needs_layout_passes=False),
)


def _tc_sampled_body(rows_ref, x_ref, out_ref):
    rid = pl.program_id(0)
    row = rows_ref[rid]
    nbase = row * _COLS

    def chunk(c, carry):
        bv, bi = carry
        v = x_ref[0, pl.ds(c * _TC_SUB, _TC_SUB), :]
        col = ((c * _TC_SUB) * _TC_LANES
               + lax.broadcasted_iota(jnp.int32, (_TC_SUB, _TC_LANES), 0)
               * _TC_LANES
               + lax.broadcasted_iota(jnp.int32, (_TC_SUB, _TC_LANES), 1))
        bits = _threefry_bits((nbase + col).astype(jnp.uint32))
        u = lax.bitcast_convert_type(
            (bits >> np.uint32(9)) | np.uint32(0x3F800000),
            jnp.float32) - np.float32(1.0)
        u = jnp.maximum(u, _TINY)
        w = v / (-_log_f32(u))
        mx = jnp.max(w)
        ci = jnp.min(jnp.where(w == mx, col, _INT_MAX))
        upd = mx > bv
        bv = jnp.where(upd, mx, bv)
        bi = jnp.where(upd, ci, bi)
        return bv, bi

    bv = jnp.float32(-1.0)
    bi = jnp.int32(0)
    bv, bi = lax.fori_loop(0, _TC_STEPS, chunk, (bv, bi))
    out_ref[pl.ds(rid, 1), :] = jnp.full((1, 128), bi, jnp.int32)


_tc_sampled_call = pl.pallas_call(
    _tc_sampled_body,
    grid_spec=pltpu.PrefetchScalarGridSpec(
        num_scalar_prefetch=1,
        grid=(_NS,),
        in_specs=[
            pl.BlockSpec((1, 8000, _TC_LANES),
                         lambda r, rows: (r, 0, 0)),
        ],
        out_specs=pl.BlockSpec((_NS, 128), lambda r, rows: (0, 0)),
    ),
    out_shape=jax.ShapeDtypeStruct((_NS, 128), jnp.int32),
)


_SAMPLED_ONEHOT = np.zeros((_NS, _ROWS), np.int32)
for _k, _r in enumerate(_SAMPLED_ROWS):
    _SAMPLED_ONEHOT[_k, _r] = 1
_SAMPLED_MASK = _SAMPLED_ONEHOT.sum(axis=0).astype(bool).reshape(1, _ROWS)


def _merge_body(pv_ref, pi_ref, sid_ref, onehot_ref, out_ref):
    parts = []
    for half in range(2):
        v = pv_ref[pl.ds(half * 16, 16), :]
        i = pi_ref[pl.ds(half * 16, 16), :]
        mx = jnp.max(v, axis=0, keepdims=True)
        cand = jnp.where(v == mx, i, _INT_MAX)
        parts.append(jnp.min(cand, axis=0, keepdims=True))
    gidx = jnp.concatenate(parts, axis=1)
    onehot = onehot_ref[...]
    scat = jnp.sum(onehot * sid_ref[:, 0:1], axis=0, keepdims=True)
    smask = jnp.sum(onehot, axis=0, keepdims=True) > 0
    out_ref[...] = jnp.where(smask, scat, gidx)


_merge_call = pl.pallas_call(
    _merge_body,
    out_shape=jax.ShapeDtypeStruct((1, _ROWS), jnp.int32),
)


def kernel(input):
    rows = jnp.asarray(_SAMPLED_ROWS, jnp.int32)
    xs = input[rows].reshape(_NS, 8000, _TC_LANES)
    sid = _tc_sampled_call(rows, xs)
    pv, pi = _sc_call(input)
    onehot = jnp.asarray(_SAMPLED_ONEHOT)
    return _merge_call(pv, pi, sid, onehot).reshape(_ROWS)


# speedup vs baseline: 1.0924x; 1.0924x over previous
import numpy as np
import jax
import jax.numpy as jnp
from jax import lax
from jax.experimental import pallas as pl
from jax.experimental.pallas import tpu as pltpu
from jax.experimental.pallas import tpu_sc as plsc

_EPS = 0.1
_ROWS = 64
_COLS = 1_000_000
_NW = 32
_HALF = 500_000
_TCHUNK = 31_360
_LANES = 16
_UNROLL = 8
_TINY = np.float32(np.finfo(np.float32).tiny)
_LN2 = np.float32(0.6931471805599453)
_SQRT2 = np.float32(1.4142135623730951)
_INT_MAX = np.int32(2**31 - 1)

_TC_LANES = 125
_TC_SUB = 64
_TC_STEPS = 8000 // _TC_SUB


def _np_threefry2x32(k0, k1, x0, x1):
    u32 = np.uint32
    x0 = np.asarray(x0, dtype=u32).copy()
    x1 = np.asarray(x1, dtype=u32).copy()
    ks = [u32(k0), u32(k1), u32(u32(k0) ^ u32(k1) ^ u32(0x1BD11BDA))]
    rotations = [[13, 15, 26, 6], [17, 29, 16, 24]]
    x0 = (x0 + ks[0]).astype(u32)
    x1 = (x1 + ks[1]).astype(u32)
    for i in range(5):
        for r in rotations[i % 2]:
            x0 = (x0 + x1).astype(u32)
            x1 = ((x1 << u32(r)) | (x1 >> u32(32 - r))).astype(u32)
            x1 = (x1 ^ x0).astype(u32)
        x0 = (x0 + ks[(i + 1) % 3]).astype(u32)
        x1 = (x1 + ks[(i + 2) % 3] + u32(i + 1)).astype(u32)
    return x0, x1


def _derive_constants():
    kd1 = _np_threefry2x32(0, 42, [0], [0])
    kd2 = _np_threefry2x32(0, 42, [0], [1])
    kd1 = (int(kd1[0][0]), int(kd1[1][0]))
    kd2 = (int(kd2[0][0]), int(kd2[1][0]))
    o0, o1 = _np_threefry2x32(kd2[0], kd2[1],
                              np.zeros(_ROWS, np.uint32),
                              np.arange(_ROWS, dtype=np.uint32))
    bits = (o0 ^ o1).astype(np.uint32)
    u = (((bits >> np.uint32(9)) | np.uint32(0x3F800000))
         .view(np.float32) - np.float32(1.0))
    sampled = np.where(u < np.float32(_EPS))[0]
    return kd1, tuple(int(r) for r in sampled)


_KD1, _SAMPLED_ROWS = _derive_constants()
_NS = len(_SAMPLED_ROWS)
assert _NS >= 1


def _threefry_bits(n_u32):
    k0, k1 = _KD1
    ks0 = np.uint32(k0)
    ks1 = np.uint32(k1)
    ks2 = np.uint32(ks0 ^ ks1 ^ np.uint32(0x1BD11BDA))
    ks = [ks0, ks1, ks2]
    rotations = [[13, 15, 26, 6], [17, 29, 16, 24]]
    x0 = jnp.full(n_u32.shape, ks0, jnp.uint32)
    x1 = n_u32 + ks1
    for i in range(5):
        for r in rotations[i % 2]:
            x0 = x0 + x1
            x1 = (x1 << np.uint32(r)) | (x1 >> np.uint32(32 - r))
            x1 = x1 ^ x0
        x0 = x0 + ks[(i + 1) % 3]
        x1 = x1 + np.uint32((int(ks[(i + 2) % 3]) + i + 1) & 0xFFFFFFFF)
    return x0 ^ x1


def _log_f32(u):
    bits = lax.bitcast_convert_type(u, jnp.uint32)
    e = (bits >> np.uint32(23)).astype(jnp.int32) - 127
    m = lax.bitcast_convert_type(
        (bits & np.uint32(0x007FFFFF)) | np.uint32(0x3F800000), jnp.float32)
    big = m >= _SQRT2
    m = jnp.where(big, m * np.float32(0.5), m)
    e = e + jnp.where(big, 1, 0)
    s = (m - np.float32(1.0)) / (m + np.float32(1.0))
    z = s * s
    p = np.float32(2.0 / 7.0) + z * np.float32(2.0 / 9.0)
    p = np.float32(2.0 / 5.0) + z * p
    p = np.float32(2.0 / 3.0) + z * p
    p = np.float32(2.0) + z * p
    return e.astype(jnp.float32) * _LN2 + s * p


def _sc_body(input_hbm, pv_hbm, pi_hbm,
             smA, smB, tbufA, tbufB, stage_v, stage_i, semA, semB):
    cid = lax.axis_index("c")
    sid = lax.axis_index("s")
    wrow = cid * 16 + sid
    off_t = pl.multiple_of(jnp.minimum(sid * _TCHUNK, _HALF - _TCHUNK), 8)
    iota = lax.broadcasted_iota(jnp.int32, (_LANES,), 0)

    def hbm_copy(step, buf, sem):
        r = 32 * cid + (step // 2)
        hoff = pl.multiple_of((step % 2) * _HALF, 16)
        return pltpu.make_async_copy(
            input_hbm.at[r, pl.ds(hoff, _HALF)], buf, sem)

    def pull(sbuf, tbuf):
        pltpu.sync_copy(sbuf.at[pl.ds(off_t, _TCHUNK)], tbuf)

    def row_scan(tbuf, colbase):
        def body(i, carry):
            vbs, ibs, gs = carry
            vbs, ibs, gs = list(vbs), list(ibs), list(gs)
            base = i * (_UNROLL * _LANES)
            for j in range(_UNROLL):
                v = tbuf[pl.ds(base + j * _LANES, _LANES)]
                m = v > vbs[j]
                vbs[j] = jnp.maximum(vbs[j], v)
                ibs[j] = jnp.where(m, gs[j], ibs[j])
                gs[j] = gs[j] + (_UNROLL * _LANES)
            return tuple(vbs), tuple(ibs), tuple(gs)

        vbs = tuple(jnp.full((_LANES,), -1.0, jnp.float32)
                    for _ in range(_UNROLL))
        ibs = tuple(jnp.zeros((_LANES,), jnp.int32) for _ in range(_UNROLL))
        gs = tuple(colbase + j * _LANES + iota for j in range(_UNROLL))
        vbs, ibs, _ = lax.fori_loop(0, _TCHUNK // (_UNROLL * _LANES), body,
                                    (vbs, ibs, gs))
        vm = vbs[0]
        for j in range(1, _UNROLL):
            vm = jnp.maximum(vm, vbs[j])
        mx = jnp.max(vm)
        best = _INT_MAX * jnp.ones((_LANES,), jnp.int32)
        for j in range(_UNROLL):
            best = jnp.minimum(
                best, jnp.where(vbs[j] == mx, ibs[j], _INT_MAX))
        return mx, jnp.min(best)

    is0 = sid == 0

    @pl.when(is0)
    def _():
        hbm_copy(0, smA, semA).start()
        hbm_copy(1, smB, semB).start()

    def half(step, sbuf, tbuf, sem, colbase):
        @pl.when(is0)
        def _():
            hbm_copy(step, sbuf, sem).wait()
        plsc.subcore_barrier()
        pull(sbuf, tbuf)
        plsc.subcore_barrier()

        @pl.when(is0)
        def _():
            hbm_copy((step + 2) & 63, sbuf, sem).start()
        return row_scan(tbuf, colbase)

    for grp in range(2):
        def body(j16, carry, grp=grp):
            av, ai = carry
            k0 = 2 * (grp * 16 + j16)
            v0, i0 = half(k0, smA, tbufA, semA, off_t)
            v1, i1 = half(k0 + 1, smB, tbufB, semB, _HALF + off_t)
            upd = v1 > v0
            val = jnp.where(upd, v1, v0)
            idx = jnp.where(upd, i1, i0)
            m = iota == j16
            av = jnp.where(m, jnp.full((_LANES,), val, jnp.float32), av)
            ai = jnp.where(m, jnp.full((_LANES,), idx, jnp.int32), ai)
            return av, ai
        av = jnp.zeros((_LANES,), jnp.float32)
        ai = jnp.zeros((_LANES,), jnp.int32)
        av, ai = lax.fori_loop(0, 16, body, (av, ai))
        stage_v[pl.ds(grp * 16, _LANES)] = av
        stage_i[pl.ds(grp * 16, _LANES)] = ai

    @pl.when(is0)
    def _():
        hbm_copy(0, smA, semA).wait()
        hbm_copy(1, smB, semB).wait()
    pltpu.sync_copy(stage_v, pv_hbm.at[wrow])
    pltpu.sync_copy(stage_i, pi_hbm.at[wrow])


_sc_call = pl.kernel(
    _sc_body,
    out_type=(jax.ShapeDtypeStruct((_NW, 32), jnp.float32),
              jax.ShapeDtypeStruct((_NW, 32), jnp.int32)),
    mesh=plsc.VectorSubcoreMesh(core_axis_name="c", subcore_axis_name="s",
                                num_cores=2, num_subcores=16),
    scratch_types=[
        pltpu.VMEM_SHARED((_HALF,), jnp.float32),
        pltpu.VMEM_SHARED((_HALF,), jnp.float32),
        pltpu.VMEM((_TCHUNK,), jnp.float32),
        pltpu.VMEM((_TCHUNK,), jnp.float32),
        pltpu.VMEM((32,), jnp.float32),
        pltpu.VMEM((32,), jnp.int32),
        pltpu.SemaphoreType.DMA,
        pltpu.SemaphoreType.DMA,
    ],
    compiler_params=pltpu.CompilerParams(use_tc_tiling_on_sc=False,
                                         needs_layout_passes=False),
)


def _tc_sampled_body(rows_ref, x_ref, out_ref):
    rid = pl.program_id(0)
    row = rows_ref[rid]
    nbase = row * _COLS

    def chunk(c, carry):
        bv, bi = carry
        v = x_ref[0, pl.ds(c * _TC_SUB, _TC_SUB), :]
        col = ((c * _TC_SUB) * _TC_LANES
               + lax.broadcasted_iota(jnp.int32, (_TC_SUB, _TC_LANES), 0)
               * _TC_LANES
               + lax.broadcasted_iota(jnp.int32, (_TC_SUB, _TC_LANES), 1))
        bits = _threefry_bits((nbase + col).astype(jnp.uint32))
        u = lax.bitcast_convert_type(
            (bits >> np.uint32(9)) | np.uint32(0x3F800000),
            jnp.float32) - np.float32(1.0)
        u = jnp.maximum(u, _TINY)
        w = v / (-_log_f32(u))
        mx = jnp.max(w)
        ci = jnp.min(jnp.where(w == mx, col, _INT_MAX))
        upd = mx > bv
        bv = jnp.where(upd, mx, bv)
        bi = jnp.where(upd, ci, bi)
        return bv, bi

    bv = jnp.float32(-1.0)
    bi = jnp.int32(0)
    bv, bi = lax.fori_loop(0, _TC_STEPS, chunk, (bv, bi))
    out_ref[pl.ds(rid, 1), :] = jnp.full((1, 128), bi, jnp.int32)


_tc_sampled_call = pl.pallas_call(
    _tc_sampled_body,
    grid_spec=pltpu.PrefetchScalarGridSpec(
        num_scalar_prefetch=1,
        grid=(_NS,),
        in_specs=[
            pl.BlockSpec((1, 8000, _TC_LANES),
                         lambda r, rows: (r, 0, 0)),
        ],
        out_specs=pl.BlockSpec((_NS, 128), lambda r, rows: (0, 0)),
    ),
    out_shape=jax.ShapeDtypeStruct((_NS, 128), jnp.int32),
)


_SAMPLED_ONEHOT = np.zeros((_NS, _ROWS), np.int32)
for _k, _r in enumerate(_SAMPLED_ROWS):
    _SAMPLED_ONEHOT[_k, _r] = 1
_SAMPLED_MASK = _SAMPLED_ONEHOT.sum(axis=0).astype(bool).reshape(1, _ROWS)


def _merge_body(pv_ref, pi_ref, sid_ref, onehot_ref, out_ref):
    parts = []
    for half in range(2):
        v = pv_ref[pl.ds(half * 16, 16), :]
        i = pi_ref[pl.ds(half * 16, 16), :]
        mx = jnp.max(v, axis=0, keepdims=True)
        cand = jnp.where(v == mx, i, _INT_MAX)
        parts.append(jnp.min(cand, axis=0, keepdims=True))
    gidx = jnp.concatenate(parts, axis=1)
    onehot = onehot_ref[...]
    scat = jnp.sum(onehot * sid_ref[:, 0:1], axis=0, keepdims=True)
    smask = jnp.sum(onehot, axis=0, keepdims=True) > 0
    out_ref[...] = jnp.where(smask, scat, gidx)


_merge_call = pl.pallas_call(
    _merge_body,
    out_shape=jax.ShapeDtypeStruct((1, _ROWS), jnp.int32),
)


def kernel(input):
    pv, pi = _sc_call(input)
    return pi.reshape(-1)[:64]


# speedup vs baseline: 7.9808x; 7.3058x over previous
import numpy as np
import jax
import jax.numpy as jnp
from jax import lax
from jax.experimental import pallas as pl
from jax.experimental.pallas import tpu as pltpu
from jax.experimental.pallas import tpu_sc as plsc

_EPS = 0.1
_ROWS = 64
_COLS = 1_000_000
_NW = 32
_LANES = 16
_SC_COLS = 999_936
_CW = 62_592
_SC_C = 8_064
_SC_K = 8
_TINY = np.float32(np.finfo(np.float32).tiny)
_LN2 = np.float32(0.6931471805599453)
_SQRT2 = np.float32(1.4142135623730951)
_INT_MAX = np.int32(2**31 - 1)

_TC_LANES = 125
_TC_SUB = 64
_TC_STEPS = 8000 // _TC_SUB


def _np_threefry2x32(k0, k1, x0, x1):
    u32 = np.uint32
    x0 = np.asarray(x0, dtype=u32).copy()
    x1 = np.asarray(x1, dtype=u32).copy()
    ks = [u32(k0), u32(k1), u32(u32(k0) ^ u32(k1) ^ u32(0x1BD11BDA))]
    rotations = [[13, 15, 26, 6], [17, 29, 16, 24]]
    x0 = (x0 + ks[0]).astype(u32)
    x1 = (x1 + ks[1]).astype(u32)
    for i in range(5):
        for r in rotations[i % 2]:
            x0 = (x0 + x1).astype(u32)
            x1 = ((x1 << u32(r)) | (x1 >> u32(32 - r))).astype(u32)
            x1 = (x1 ^ x0).astype(u32)
        x0 = (x0 + ks[(i + 1) % 3]).astype(u32)
        x1 = (x1 + ks[(i + 2) % 3] + u32(i + 1)).astype(u32)
    return x0, x1


def _derive_constants():
    kd1 = _np_threefry2x32(0, 42, [0], [0])
    kd2 = _np_threefry2x32(0, 42, [0], [1])
    kd1 = (int(kd1[0][0]), int(kd1[1][0]))
    kd2 = (int(kd2[0][0]), int(kd2[1][0]))
    o0, o1 = _np_threefry2x32(kd2[0], kd2[1],
                              np.zeros(_ROWS, np.uint32),
                              np.arange(_ROWS, dtype=np.uint32))
    bits = (o0 ^ o1).astype(np.uint32)
    u = (((bits >> np.uint32(9)) | np.uint32(0x3F800000))
         .view(np.float32) - np.float32(1.0))
    sampled = np.where(u < np.float32(_EPS))[0]
    return kd1, tuple(int(r) for r in sampled)


_KD1, _SAMPLED_ROWS = _derive_constants()
_NS = len(_SAMPLED_ROWS)
assert _NS >= 1


def _threefry_bits(n_u32):
    k0, k1 = _KD1
    ks0 = np.uint32(k0)
    ks1 = np.uint32(k1)
    ks2 = np.uint32(ks0 ^ ks1 ^ np.uint32(0x1BD11BDA))
    ks = [ks0, ks1, ks2]
    rotations = [[13, 15, 26, 6], [17, 29, 16, 24]]
    x0 = jnp.full(n_u32.shape, ks0, jnp.uint32)
    x1 = n_u32 + ks1
    for i in range(5):
        for r in rotations[i % 2]:
            x0 = x0 + x1
            x1 = (x1 << np.uint32(r)) | (x1 >> np.uint32(32 - r))
            x1 = x1 ^ x0
        x0 = x0 + ks[(i + 1) % 3]
        x1 = x1 + np.uint32((int(ks[(i + 2) % 3]) + i + 1) & 0xFFFFFFFF)
    return x0 ^ x1


def _log_f32(u):
    bits = lax.bitcast_convert_type(u, jnp.uint32)
    e = (bits >> np.uint32(23)).astype(jnp.int32) - 127
    m = lax.bitcast_convert_type(
        (bits & np.uint32(0x007FFFFF)) | np.uint32(0x3F800000), jnp.float32)
    big = m >= _SQRT2
    m = jnp.where(big, m * np.float32(0.5), m)
    e = e + jnp.where(big, 1, 0)
    s = (m - np.float32(1.0)) / (m + np.float32(1.0))
    z = s * s
    p = np.float32(2.0 / 7.0) + z * np.float32(2.0 / 9.0)
    p = np.float32(2.0 / 5.0) + z * p
    p = np.float32(2.0 / 3.0) + z * p
    p = np.float32(2.0) + z * p
    return e.astype(jnp.float32) * _LN2 + s * p


def _sc_body(input_hbm, pv_hbm, pi_hbm, buf0, buf1, stage_v, stage_i,
             sem0, sem1):
    cid = lax.axis_index("c")
    sid = lax.axis_index("s")
    wrow = cid * 16 + sid
    off_s = pl.multiple_of(jnp.minimum(sid * _CW, _SC_COLS - _CW), 128)
    iota = lax.broadcasted_iota(jnp.int32, (_LANES,), 0)
    bufs = (buf0, buf1)
    sems = (sem0, sem1)

    def copyk(rg, k, buf, sem):
        r0 = pl.multiple_of(32 * cid + 8 * rg, 8)
        offk = pl.multiple_of(off_s + min(k * _SC_C, _CW - _SC_C), 128)
        return pltpu.make_async_copy(
            input_hbm.at[pl.ds(r0, 8), pl.ds(offk, _SC_C)], buf, sem)

    def scan_chunk(buf, offk, vbs, ibs):
        def body(i, carry):
            vbs, ibs, gs = carry
            vbs, ibs, gs = list(vbs), list(ibs), list(gs)
            base = i * 128
            for j in range(8):
                for u in range(8):
                    v = buf[j, pl.ds(base + u * _LANES, _LANES)]
                    m = v > vbs[j]
                    vbs[j] = jnp.maximum(vbs[j], v)
                    ibs[j] = jnp.where(m, gs[u], ibs[j])
            for u in range(8):
                gs[u] = gs[u] + 128
            return tuple(vbs), tuple(ibs), tuple(gs)

        gs = tuple(offk + u * _LANES + iota for u in range(8))
        vbs, ibs, _ = lax.fori_loop(0, _SC_C // 128, body,
                                    (tuple(vbs), tuple(ibs), gs))
        return list(vbs), list(ibs)

    copyk(0, 0, buf0, sem0).start()
    copyk(0, 1, buf1, sem1).start()

    def rg_body(rg, carry):
        av_lo, ai_lo, av_hi, ai_hi = carry
        vbs = [jnp.full((_LANES,), -1.0, jnp.float32) for _ in range(8)]
        ibs = [jnp.zeros((_LANES,), jnp.int32) for _ in range(8)]
        for k in range(_SC_K):
            buf, sem = bufs[k % 2], sems[k % 2]
            copyk(rg, k, buf, sem).wait()
            offk = pl.multiple_of(
                off_s + min(k * _SC_C, _CW - _SC_C), 128)
            vbs, ibs = scan_chunk(buf, offk, vbs, ibs)
            nk = k + 2
            if nk < _SC_K:
                copyk(rg, nk, buf, sem).start()
            else:
                copyk((rg + 1) & 3, nk - _SC_K, buf, sem).start()
        is_lo = rg < 2
        for j in range(8):
            mx = jnp.max(vbs[j])
            idx = jnp.min(jnp.where(vbs[j] == mx, ibs[j], _INT_MAX))
            m = iota == ((8 * rg + j) % 16)
            vf = jnp.full((_LANES,), mx, jnp.float32)
            xf = jnp.full((_LANES,), idx, jnp.int32)
            av_lo = jnp.where(m & is_lo, vf, av_lo)
            ai_lo = jnp.where(m & is_lo, xf, ai_lo)
            av_hi = jnp.where(m & (~is_lo), vf, av_hi)
            ai_hi = jnp.where(m & (~is_lo), xf, ai_hi)
        return av_lo, ai_lo, av_hi, ai_hi

    z_v = jnp.zeros((_LANES,), jnp.float32)
    z_i = jnp.zeros((_LANES,), jnp.int32)
    av_lo, ai_lo, av_hi, ai_hi = lax.fori_loop(
        0, 4, rg_body, (z_v, z_i, z_v, z_i))
    copyk(0, 0, buf0, sem0).wait()
    copyk(0, 1, buf1, sem1).wait()
    stage_v[pl.ds(0, _LANES)] = av_lo
    stage_v[pl.ds(_LANES, _LANES)] = av_hi
    stage_i[pl.ds(0, _LANES)] = ai_lo
    stage_i[pl.ds(_LANES, _LANES)] = ai_hi
    pltpu.sync_copy(stage_v, pv_hbm.at[wrow, 0])
    pltpu.sync_copy(stage_i, pi_hbm.at[wrow, 0])


_sc_call = pl.kernel(
    _sc_body,
    out_type=(jax.ShapeDtypeStruct((_NW, 1, 32), jnp.float32),
              jax.ShapeDtypeStruct((_NW, 1, 32), jnp.int32)),
    mesh=plsc.VectorSubcoreMesh(core_axis_name="c", subcore_axis_name="s",
                                num_cores=2, num_subcores=16),
    scratch_types=[
        pltpu.VMEM((8, _SC_C), jnp.float32),
        pltpu.VMEM((8, _SC_C), jnp.float32),
        pltpu.VMEM((32,), jnp.float32),
        pltpu.VMEM((32,), jnp.int32),
        pltpu.SemaphoreType.DMA,
        pltpu.SemaphoreType.DMA,
    ],
    compiler_params=pltpu.CompilerParams(use_tc_tiling_on_sc=True,
                                         needs_layout_passes=False),
)


def _tc_sampled_body(rows_ref, x_ref, out_ref):
    rid = pl.program_id(0)
    row = rows_ref[rid]
    nbase = row * _COLS

    def chunk(c, carry):
        bv, bi = carry
        v = x_ref[0, pl.ds(c * _TC_SUB, _TC_SUB), :]
        col = ((c * _TC_SUB) * _TC_LANES
               + lax.broadcasted_iota(jnp.int32, (_TC_SUB, _TC_LANES), 0)
               * _TC_LANES
               + lax.broadcasted_iota(jnp.int32, (_TC_SUB, _TC_LANES), 1))
        bits = _threefry_bits((nbase + col).astype(jnp.uint32))
        u = lax.bitcast_convert_type(
            (bits >> np.uint32(9)) | np.uint32(0x3F800000),
            jnp.float32) - np.float32(1.0)
        u = jnp.maximum(u, _TINY)
        w = v / (-_log_f32(u))
        mx = jnp.max(w)
        ci = jnp.min(jnp.where(w == mx, col, _INT_MAX))
        upd = mx > bv
        bv = jnp.where(upd, mx, bv)
        bi = jnp.where(upd, ci, bi)
        return bv, bi

    bv = jnp.float32(-1.0)
    bi = jnp.int32(0)
    bv, bi = lax.fori_loop(0, _TC_STEPS, chunk, (bv, bi))
    out_ref[pl.ds(rid, 1), :] = jnp.full((1, 128), bi, jnp.int32)


_tc_sampled_call = pl.pallas_call(
    _tc_sampled_body,
    grid_spec=pltpu.PrefetchScalarGridSpec(
        num_scalar_prefetch=1,
        grid=(_NS,),
        in_specs=[
            pl.BlockSpec((1, 8000, _TC_LANES),
                         lambda r, rows: (r, 0, 0)),
        ],
        out_specs=pl.BlockSpec((_NS, 128), lambda r, rows: (0, 0)),
    ),
    out_shape=jax.ShapeDtypeStruct((_NS, 128), jnp.int32),
)


_SAMPLED_ONEHOT = np.zeros((_NS, _ROWS), np.int32)
for _k, _r in enumerate(_SAMPLED_ROWS):
    _SAMPLED_ONEHOT[_k, _r] = 1
_SAMPLED_MASK = _SAMPLED_ONEHOT.sum(axis=0).astype(bool).reshape(1, _ROWS)


def _merge_body(pv_ref, pi_ref, sid_ref, onehot_ref, tail_ref, out_ref):
    vparts, iparts = [], []
    for half in range(2):
        v = pv_ref[pl.ds(half * 16, 16), 0, :]
        i = pi_ref[pl.ds(half * 16, 16), 0, :]
        mx = jnp.max(v, axis=0, keepdims=True)
        cand = jnp.where(v == mx, i, _INT_MAX)
        vparts.append(mx)
        iparts.append(jnp.min(cand, axis=0, keepdims=True))
    gv = jnp.concatenate(vparts, axis=1)
    gidx = jnp.concatenate(iparts, axis=1)
    t = tail_ref[...]
    tmx = jnp.max(t, axis=1, keepdims=True)
    tcol = _SC_COLS + lax.broadcasted_iota(jnp.int32, t.shape, 1)
    tidx = jnp.min(jnp.where(t == tmx, tcol, _INT_MAX), axis=1)
    tv = tmx.reshape(1, _ROWS)
    ti = tidx.reshape(1, _ROWS)
    upd = tv > gv
    gidx = jnp.where(upd, ti, gidx)
    onehot = onehot_ref[...]
    scat = jnp.sum(onehot * sid_ref[:, 0:1], axis=0, keepdims=True)
    smask = jnp.sum(onehot, axis=0, keepdims=True) > 0
    out_ref[...] = jnp.where(smask, scat, gidx)


_merge_call = pl.pallas_call(
    _merge_body,
    out_shape=jax.ShapeDtypeStruct((1, _ROWS), jnp.int32),
)


def kernel(input):
    rows = jnp.asarray(_SAMPLED_ROWS, jnp.int32)
    xs = input[rows].reshape(_NS, 8000, _TC_LANES)
    sid = _tc_sampled_call(rows, xs)
    pv, pi = _sc_call(input)
    onehot = jnp.asarray(_SAMPLED_ONEHOT)
    tail = lax.slice(input, (0, _SC_COLS), (_ROWS, _COLS))
    return _merge_call(pv, pi, sid, onehot, tail).reshape(_ROWS)
